# Initial kernel scaffold; baseline (speedup 1.0000x reference)
import jax
import jax.numpy as jnp
from jax.experimental import pallas as pl


def kernel(raw_data, emb_value, emb_alpha, factor):
    raise NotImplementedError("write your pallas kernel here")


# speedup vs baseline: 201.0946x; 201.0946x over previous
import functools

import jax
import jax.numpy as jnp
from jax import lax
from jax.experimental import pallas as pl
from jax.experimental.pallas import tpu as pltpu
from jax.experimental.pallas import tpu_sc as plsc

_INPUT_DIM = 4096
_NUM_WORKERS = 32
_CHUNK = 16384


def kernel(raw_data, emb_value, emb_alpha, factor):
    B, C, D, H, W = raw_data.shape
    DHW = D * H * W
    N = B * C * DHW
    per_w = N // _NUM_WORKERS
    steps = per_w // _CHUNK

    raw_flat = raw_data.reshape(N)
    vtab_host = jnp.clip(emb_value.reshape(-1), 0.0, 1.0)
    atab_host = jnp.clip(emb_alpha.reshape(-1), 0.0, 1.0) * jnp.float32(factor)

    mesh = plsc.VectorSubcoreMesh(core_axis_name="c", subcore_axis_name="s")

    @functools.partial(
        pl.kernel,
        mesh=mesh,
        compiler_params=pltpu.CompilerParams(needs_layout_passes=False),
        out_type=jax.ShapeDtypeStruct((B, 4, DHW), jnp.float32),
        scratch_types=[
            pltpu.VMEM((_INPUT_DIM,), jnp.float32),
            pltpu.VMEM((_INPUT_DIM,), jnp.float32),
            pltpu.VMEM((_CHUNK,), jnp.float32),
            pltpu.VMEM((_CHUNK,), jnp.float32),
            pltpu.VMEM((_CHUNK,), jnp.float32),
        ],
    )
    def _lut_kernel(raw_hbm, vtab_hbm, atab_hbm, out_hbm, vtab, atab, rawb,
                    vbuf, abuf):
        wid = lax.axis_index("s") * 2 + lax.axis_index("c")
        pltpu.sync_copy(vtab_hbm, vtab)
        pltpu.sync_copy(atab_hbm, atab)
        base = wid * per_w

        def step(i, carry):
            off = base + i * _CHUNK
            pltpu.sync_copy(raw_hbm.at[pl.ds(off, _CHUNK)], rawb)

            def inner(j, carry2):
                x = rawb[pl.ds(j * 16, 16)]
                idx = (x * (_INPUT_DIM - 1)).astype(jnp.int32)
                vbuf[pl.ds(j * 16, 16)] = plsc.load_gather(vtab, [idx])
                abuf[pl.ds(j * 16, 16)] = plsc.load_gather(atab, [idx])
                return carry2

            lax.fori_loop(0, _CHUNK // 16, inner, 0, unroll=4)
            b = off // DHW
            o = off - b * DHW
            pltpu.sync_copy(vbuf, out_hbm.at[b, 0, pl.ds(o, _CHUNK)])
            pltpu.sync_copy(vbuf, out_hbm.at[b, 1, pl.ds(o, _CHUNK)])
            pltpu.sync_copy(vbuf, out_hbm.at[b, 2, pl.ds(o, _CHUNK)])
            pltpu.sync_copy(abuf, out_hbm.at[b, 3, pl.ds(o, _CHUNK)])
            return carry

        lax.fori_loop(0, steps, step, 0)

    out = _lut_kernel(raw_flat, vtab_host, atab_host)
    return out.reshape(B, 4, D, H, W)


# speedup vs baseline: 221.9897x; 1.1039x over previous
import functools

import jax
import jax.numpy as jnp
from jax import lax
from jax.experimental import pallas as pl
from jax.experimental.pallas import tpu as pltpu
from jax.experimental.pallas import tpu_sc as plsc

_INPUT_DIM = 4096
_NUM_WORKERS = 32
_CHUNK = 16384


def kernel(raw_data, emb_value, emb_alpha, factor):
    B, C, D, H, W = raw_data.shape
    DHW = D * H * W
    N = B * C * DHW
    per_w = N // _NUM_WORKERS
    steps = per_w // _CHUNK

    raw_flat = raw_data.reshape(N)
    vtab_host = jnp.clip(emb_value.reshape(-1), 0.0, 1.0)
    atab_host = jnp.clip(emb_alpha.reshape(-1), 0.0, 1.0) * jnp.float32(factor)

    mesh = plsc.VectorSubcoreMesh(core_axis_name="c", subcore_axis_name="s")

    @functools.partial(
        pl.kernel,
        mesh=mesh,
        compiler_params=pltpu.CompilerParams(needs_layout_passes=False),
        out_type=jax.ShapeDtypeStruct((B, 4, DHW), jnp.float32),
        scratch_types=[
            pltpu.VMEM((_INPUT_DIM,), jnp.float32),
            pltpu.VMEM((_INPUT_DIM,), jnp.float32),
            pltpu.VMEM((2, _CHUNK), jnp.float32),
            pltpu.VMEM((2, _CHUNK), jnp.float32),
            pltpu.VMEM((2, _CHUNK), jnp.float32),
            pltpu.SemaphoreType.DMA((2,)),
            pltpu.SemaphoreType.DMA((2,)),
        ],
    )
    def _lut_kernel(raw_hbm, vtab_hbm, atab_hbm, out_hbm, vtab, atab, rawb,
                    vbuf, abuf, in_sem, out_sem):
        wid = lax.axis_index("s") * 2 + lax.axis_index("c")
        pltpu.sync_copy(vtab_hbm, vtab)
        pltpu.sync_copy(atab_hbm, atab)
        base = wid * per_w

        def start_in(g, slot):
            off = base + g * _CHUNK
            return pltpu.async_copy(
                raw_hbm.at[pl.ds(off, _CHUNK)], rawb.at[slot], in_sem.at[slot]
            )

        def start_out(g, slot):
            off = base + g * _CHUNK
            b = off // DHW
            o = off - b * DHW
            return [
                pltpu.async_copy(
                    vbuf.at[slot], out_hbm.at[b, c, pl.ds(o, _CHUNK)],
                    out_sem.at[slot],
                )
                for c in range(3)
            ] + [
                pltpu.async_copy(
                    abuf.at[slot], out_hbm.at[b, 3, pl.ds(o, _CHUNK)],
                    out_sem.at[slot],
                )
            ]

        in_handles = [None, None]
        out_handles = [None, None]
        in_handles[0] = start_in(0, 0)
        if steps > 1:
            in_handles[1] = start_in(1, 1)

        for g in range(steps):
            slot = g % 2
            in_handles[slot].wait()
            if out_handles[slot] is not None:
                for h in out_handles[slot]:
                    h.wait()

            def inner(j, carry, slot=slot):
                x = rawb[slot, pl.ds(j * 16, 16)]
                idx = (x * (_INPUT_DIM - 1)).astype(jnp.int32)
                vbuf[slot, pl.ds(j * 16, 16)] = plsc.load_gather(vtab, [idx])
                abuf[slot, pl.ds(j * 16, 16)] = plsc.load_gather(atab, [idx])
                return carry

            lax.fori_loop(0, _CHUNK // 16, inner, 0, unroll=4)

            if g + 2 < steps:
                in_handles[slot] = start_in(g + 2, slot)
            out_handles[slot] = start_out(g, slot)

        for hs in out_handles:
            if hs is not None:
                for h in hs:
                    h.wait()

    out = _lut_kernel(raw_flat, vtab_host, atab_host)
    return out.reshape(B, 4, D, H, W)


# speedup vs baseline: 301.5461x; 1.3584x over previous
import functools

import jax
import jax.numpy as jnp
from jax import lax
from jax.experimental import pallas as pl
from jax.experimental.pallas import tpu as pltpu
from jax.experimental.pallas import tpu_sc as plsc

_INPUT_DIM = 4096
_NUM_WORKERS = 32
_CHUNK = 16384


def kernel(raw_data, emb_value, emb_alpha, factor):
    B, C, D, H, W = raw_data.shape
    DHW = D * H * W
    N = B * C * DHW
    per_w = N // _NUM_WORKERS
    steps = per_w // _CHUNK

    raw_flat = raw_data.reshape(N)
    vtab_host = jnp.clip(emb_value.reshape(-1), 0.0, 1.0)
    atab_host = jnp.clip(emb_alpha.reshape(-1), 0.0, 1.0) * jnp.float32(factor)

    mesh = plsc.VectorSubcoreMesh(core_axis_name="c", subcore_axis_name="s")

    @functools.partial(
        pl.kernel,
        mesh=mesh,
        compiler_params=pltpu.CompilerParams(needs_layout_passes=False),
        out_type=jax.ShapeDtypeStruct((B, 4, DHW), jnp.float32),
        scratch_types=[
            pltpu.VMEM((_INPUT_DIM,), jnp.float32),
            pltpu.VMEM((_INPUT_DIM,), jnp.float32),
            pltpu.VMEM((2, _CHUNK), jnp.float32),
            pltpu.VMEM((2, _CHUNK), jnp.float32),
            pltpu.VMEM((2, _CHUNK), jnp.float32),
            pltpu.SemaphoreType.DMA((2,)),
            pltpu.SemaphoreType.DMA((2,)),
        ],
    )
    def _lut_kernel(raw_hbm, vtab_hbm, atab_hbm, out_hbm, vtab, atab, rawb,
                    vbuf, abuf, in_sem, out_sem):
        wid = lax.axis_index("s") * 2 + lax.axis_index("c")
        pltpu.sync_copy(vtab_hbm, vtab)
        pltpu.sync_copy(atab_hbm, atab)
        base = wid * per_w

        def start_in(g, slot):
            off = base + g * _CHUNK
            return pltpu.async_copy(
                raw_hbm.at[pl.ds(off, _CHUNK)], rawb.at[slot], in_sem.at[slot]
            )

        def start_out(g, slot):
            off = base + g * _CHUNK
            b = off // DHW
            o = off - b * DHW
            return [
                pltpu.async_copy(
                    vbuf.at[slot], out_hbm.at[b, c, pl.ds(o, _CHUNK)],
                    out_sem.at[slot],
                )
                for c in range(3)
            ] + [
                pltpu.async_copy(
                    abuf.at[slot], out_hbm.at[b, 3, pl.ds(o, _CHUNK)],
                    out_sem.at[slot],
                )
            ]

        in_handles = [None, None]
        out_handles = [None, None]
        in_handles[0] = start_in(0, 0)
        if steps > 1:
            in_handles[1] = start_in(1, 1)

        for g in range(steps):
            slot = g % 2
            in_handles[slot].wait()
            if out_handles[slot] is not None:
                for h in out_handles[slot]:
                    h.wait()

            @plsc.parallel_loop(0, _CHUNK, 16, unroll=8)
            def inner(j, slot=slot):
                x = rawb[slot, pl.ds(j, 16)]
                idx = (x * (_INPUT_DIM - 1)).astype(jnp.int32)
                vbuf[slot, pl.ds(j, 16)] = plsc.load_gather(vtab, [idx])
                abuf[slot, pl.ds(j, 16)] = plsc.load_gather(atab, [idx])

            if g + 2 < steps:
                in_handles[slot] = start_in(g + 2, slot)
            out_handles[slot] = start_out(g, slot)

        for hs in out_handles:
            if hs is not None:
                for h in hs:
                    h.wait()

    out = _lut_kernel(raw_flat, vtab_host, atab_host)
    return out.reshape(B, 4, D, H, W)


# speedup vs baseline: 1301.2579x; 4.3153x over previous
import functools

import jax
import jax.numpy as jnp
from jax import lax
from jax.experimental import pallas as pl
from jax.experimental.pallas import tpu as pltpu
from jax.experimental.pallas import tpu_sc as plsc

_INPUT_DIM = 4096
_NUM_WORKERS = 32


def kernel(raw_data, emb_value, emb_alpha, factor):
    B, C, D, H, W = raw_data.shape
    n_planes = B * C * D
    steps = n_planes // _NUM_WORKERS
    col_chunks = W // 16

    raw_planes = raw_data.reshape(n_planes, H, W)
    vtab_host = jnp.clip(emb_value.reshape(-1), 0.0, 1.0)
    atab_host = jnp.clip(emb_alpha.reshape(-1), 0.0, 1.0) * jnp.float32(factor)

    mesh = plsc.VectorSubcoreMesh(core_axis_name="c", subcore_axis_name="s")

    @functools.partial(
        pl.kernel,
        mesh=mesh,
        compiler_params=pltpu.CompilerParams(needs_layout_passes=False),
        out_type=jax.ShapeDtypeStruct((B, 4, D, H, W), jnp.float32),
        scratch_types=[
            pltpu.VMEM((_INPUT_DIM,), jnp.float32),
            pltpu.VMEM((_INPUT_DIM,), jnp.float32),
            pltpu.VMEM((2, H, W), jnp.float32),
            pltpu.VMEM((2, H, W), jnp.float32),
            pltpu.VMEM((2, H, W), jnp.float32),
            pltpu.SemaphoreType.DMA((2,)),
            pltpu.SemaphoreType.DMA((2,)),
        ],
    )
    def _lut_kernel(raw_hbm, vtab_hbm, atab_hbm, out_hbm, vtab, atab, rawb,
                    vbuf, abuf, in_sem, out_sem):
        wid = lax.axis_index("s") * 2 + lax.axis_index("c")
        pltpu.sync_copy(vtab_hbm, vtab)
        pltpu.sync_copy(atab_hbm, atab)
        base = wid * steps

        def start_in(g, slot):
            return pltpu.async_copy(
                raw_hbm.at[base + g], rawb.at[slot], in_sem.at[slot]
            )

        def start_out(g, slot):
            p = base + g
            b = p // D
            dpl = p - b * D
            return [
                pltpu.async_copy(
                    vbuf.at[slot], out_hbm.at[b, c, dpl], out_sem.at[slot]
                )
                for c in range(3)
            ] + [
                pltpu.async_copy(
                    abuf.at[slot], out_hbm.at[b, 3, dpl], out_sem.at[slot]
                )
            ]

        in_handles = [None, None]
        out_handles = [None, None]
        in_handles[0] = start_in(0, 0)
        if steps > 1:
            in_handles[1] = start_in(1, 1)

        for g in range(steps):
            slot = g % 2
            in_handles[slot].wait()
            if out_handles[slot] is not None:
                for h in out_handles[slot]:
                    h.wait()

            @plsc.parallel_loop(0, H, 1, unroll=2)
            def inner(r, slot=slot):
                for cc in range(col_chunks):
                    x = rawb[slot, r, pl.ds(cc * 16, 16)]
                    idx = (x * (_INPUT_DIM - 1)).astype(jnp.int32)
                    vbuf[slot, r, pl.ds(cc * 16, 16)] = plsc.load_gather(
                        vtab, [idx]
                    )
                    abuf[slot, r, pl.ds(cc * 16, 16)] = plsc.load_gather(
                        atab, [idx]
                    )

            if g + 2 < steps:
                in_handles[slot] = start_in(g + 2, slot)
            out_handles[slot] = start_out(g, slot)

        for hs in out_handles:
            if hs is not None:
                for h in hs:
                    h.wait()

    return _lut_kernel(raw_planes, vtab_host, atab_host)
